# Initial kernel scaffold; baseline (speedup 1.0000x reference)
import jax
import jax.numpy as jnp
from jax import lax
from jax.experimental import pallas as pl
from jax.experimental.pallas import tpu as pltpu
from jax.experimental.pallas import tpu_sc as plsc

N = 10000
E = 320000
D = 128
G = 64
NP = 10240
NC = 2
NS = 16
NWK = NC * NS
EPW = E // NWK
K = 128
NWIN = EPW // K
REM = EPW - NWIN * K
TPS = NP // NS

_mesh = plsc.VectorSubcoreMesh(core_axis_name="c", subcore_axis_name="s")


def _deg_body(dst_hbm, out_hbm, idx_v, idxr_v, ones_v, zb_v, acc_sh):
    c = lax.axis_index("c")
    s = lax.axis_index("s")
    w = s * NC + c

    def fill_ones(i, carry):
        ones_v[pl.ds(i * 16, 16)] = jnp.ones((16,), jnp.float32)
        return carry

    lax.fori_loop(0, K // 16, fill_ones, 0)

    def fill_zeros(i, carry):
        zb_v[pl.ds(i * 16, 16)] = jnp.zeros((16,), jnp.float32)
        return carry

    lax.fori_loop(0, TPS // 16, fill_zeros, 0)
    pltpu.sync_copy(zb_v, acc_sh.at[pl.ds(s * TPS, TPS)])
    plsc.subcore_barrier()

    base0 = w * EPW

    def win(j, carry):
        pltpu.sync_copy(dst_hbm.at[pl.ds(base0 + j * K, K)], idx_v)
        pltpu.sync_copy(ones_v, acc_sh.at[idx_v], add=True)
        return carry

    lax.fori_loop(0, NWIN, win, 0)
    pltpu.sync_copy(dst_hbm.at[pl.ds(base0 + NWIN * K, REM)], idxr_v)
    pltpu.sync_copy(ones_v.at[pl.ds(0, REM)], acc_sh.at[idxr_v], add=True)
    plsc.subcore_barrier()
    pltpu.sync_copy(acc_sh.at[pl.ds(s * TPS, TPS)],
                    out_hbm.at[pl.ds(c * NP + s * TPS, TPS)])


_deg_call = pl.kernel(
    _deg_body,
    out_type=jax.ShapeDtypeStruct((NC * NP,), jnp.float32),
    mesh=_mesh,
    scratch_types=[
        pltpu.VMEM((K,), jnp.int32),
        pltpu.VMEM((REM,), jnp.int32),
        pltpu.VMEM((K,), jnp.float32),
        pltpu.VMEM((TPS,), jnp.float32),
        pltpu.VMEM_SHARED((NP,), jnp.float32),
    ],
)


def _agg_body(src_hbm, dst_hbm, y_hbm, out_hbm,
              si_v, di_v, sir_v, dir_v, rows_v, rowsr_v, zb_v, acc_sh, sem):
    c = lax.axis_index("c")
    s = lax.axis_index("s")
    w = s * NC + c

    def fill_zeros(i, carry):
        zb_v[i // 8, pl.ds((i % 8) * 16, 16)] = jnp.zeros((16,), jnp.float32)
        return carry

    lax.fori_loop(0, (128 * D) // 16, fill_zeros, 0)

    def zcp(t, carry):
        pltpu.sync_copy(zb_v, acc_sh.at[pl.ds(s * TPS + t * 128, 128)])
        return carry

    lax.fori_loop(0, TPS // 128, zcp, 0)
    plsc.subcore_barrier()

    base0 = w * EPW

    def win(j, carry):
        base = base0 + j * K
        pltpu.sync_copy(src_hbm.at[pl.ds(base, K)], si_v)
        pltpu.sync_copy(dst_hbm.at[pl.ds(base, K)], di_v)
        pltpu.async_copy(y_hbm.at[si_v], rows_v, sem).wait()
        pltpu.sync_copy(rows_v, acc_sh.at[di_v], add=True)
        return carry

    lax.fori_loop(0, NWIN, win, 0)

    base = base0 + NWIN * K
    pltpu.sync_copy(src_hbm.at[pl.ds(base, REM)], sir_v)
    pltpu.sync_copy(dst_hbm.at[pl.ds(base, REM)], dir_v)
    pltpu.async_copy(y_hbm.at[sir_v], rowsr_v, sem).wait()
    pltpu.sync_copy(rowsr_v, acc_sh.at[dir_v], add=True)

    plsc.subcore_barrier()
    pltpu.sync_copy(acc_sh.at[pl.ds(s * TPS, TPS)],
                    out_hbm.at[pl.ds(c * NP + s * TPS, TPS)])


_agg_call = pl.kernel(
    _agg_body,
    out_type=jax.ShapeDtypeStruct((NC * NP, D), jnp.float32),
    mesh=_mesh,
    scratch_types=[
        pltpu.VMEM((K,), jnp.int32),
        pltpu.VMEM((K,), jnp.int32),
        pltpu.VMEM((REM,), jnp.int32),
        pltpu.VMEM((REM,), jnp.int32),
        pltpu.VMEM((K, D), jnp.float32),
        pltpu.VMEM((REM, D), jnp.float32),
        pltpu.VMEM((128, D), jnp.float32),
        pltpu.VMEM_SHARED((NP, D), jnp.float32),
        pltpu.SemaphoreType.DMA,
    ],
)

R = 2000
GRID = N // R


def _tc_first_body(x_ref, w_ref, dis_ref, y_ref):
    y_ref[...] = dis_ref[...] * jnp.dot(
        x_ref[...], w_ref[...], preferred_element_type=jnp.float32)


def _tc_mid_body(a0_ref, a1_ref, yp_ref, dis_ref, b_ref, w_ref, y_ref):
    dis = dis_ref[...]
    h = jnp.tanh(dis * (a0_ref[0] + a1_ref[0] + yp_ref[...]) + b_ref[...])
    y_ref[...] = dis * jnp.dot(h, w_ref[...],
                               preferred_element_type=jnp.float32)


def _tc_final_body(a0_ref, a1_ref, yp_ref, dis_ref, b_ref, wl_ref, bl_ref,
                   batch_ref, out_ref):
    i = pl.program_id(0)
    dis = dis_ref[...]
    h = jnp.tanh(dis * (a0_ref[0] + a1_ref[0] + yp_ref[...]) + b_ref[...])
    t = jnp.tanh(jnp.dot(h, wl_ref[...],
                         preferred_element_type=jnp.float32) + bl_ref[...])
    lane = lax.broadcasted_iota(jnp.int32, (R, D), 1)
    t = jnp.where(lane == 6, 1.0, t)
    oneh = (batch_ref[...] == lane).astype(jnp.float32)
    acc = lax.dot_general(oneh, t, (((0,), (0,)), ((), ())),
                          preferred_element_type=jnp.float32)

    @pl.when(i == 0)
    def _init():
        out_ref[...] = acc

    @pl.when(i > 0)
    def _accum():
        out_ref[...] += acc


_row_spec = pl.BlockSpec((R, D), lambda i: (i, 0))
_w_spec = pl.BlockSpec((D, D), lambda i: (0, 0))
_b_spec = pl.BlockSpec((1, D), lambda i: (0, 0))
_a0_spec = pl.BlockSpec((1, R, D), lambda i: (0, i, 0))
_a1_spec = pl.BlockSpec((1, R, D), lambda i: (1, i, 0))

_tc_first = pl.pallas_call(
    _tc_first_body, grid=(GRID,),
    in_specs=[_row_spec, _w_spec, _row_spec],
    out_specs=_row_spec,
    out_shape=jax.ShapeDtypeStruct((N, D), jnp.float32),
)

_tc_mid = pl.pallas_call(
    _tc_mid_body, grid=(GRID,),
    in_specs=[_a0_spec, _a1_spec, _row_spec, _row_spec, _b_spec, _w_spec],
    out_specs=_row_spec,
    out_shape=jax.ShapeDtypeStruct((N, D), jnp.float32),
)

_tc_final = pl.pallas_call(
    _tc_final_body, grid=(GRID,),
    in_specs=[_a0_spec, _a1_spec, _row_spec, _row_spec, _b_spec, _w_spec,
              _b_spec, _row_spec],
    out_specs=pl.BlockSpec((D, D), lambda i: (0, 0)),
    out_shape=jax.ShapeDtypeStruct((D, D), jnp.float32),
)


def kernel(x, edge_index, batch, W0, b0, W1, b1, W2, b2, Wl, bl):
    src = edge_index[0]
    dst = edge_index[1]

    degp = _deg_call(dst)
    deg = 1.0 + degp[:N] + degp[NP:NP + N]
    dis = lax.rsqrt(deg)
    dis_b = jnp.broadcast_to(dis[:, None], (N, D))

    y0 = _tc_first(x, W0, dis_b)
    a0 = _agg_call(src, dst, y0).reshape(NC, NP, D)
    y1 = _tc_mid(a0, a0, y0, dis_b, b0.reshape(1, D), W1)
    a1 = _agg_call(src, dst, y1).reshape(NC, NP, D)
    y2 = _tc_mid(a1, a1, y1, dis_b, b1.reshape(1, D), W2)
    a2 = _agg_call(src, dst, y2).reshape(NC, NP, D)

    Wlp = jnp.pad(Wl, ((0, 0), (0, D - 6)))
    blp = jnp.pad(bl, (0, D - 6)).reshape(1, D)
    batch_b = jnp.broadcast_to(batch[:, None], (N, D))
    P = _tc_final(a2, a2, y2, dis_b, b2.reshape(1, D), Wlp, blp, batch_b)

    sums = P[:G, :6]
    counts = P[:G, 6]
    return sums / jnp.clip(counts, 1.0)[:, None]


# speedup vs baseline: 14.3288x; 14.3288x over previous
import functools

import jax
import jax.numpy as jnp
from jax import lax
from jax.experimental import pallas as pl
from jax.experimental.pallas import tpu as pltpu
from jax.experimental.pallas import tpu_sc as plsc

N = 10000
E = 320000
D = 128
G = 64
NP = 10240
NC = 2
NS = 16
NWK = NC * NS
EPW = E // NWK
K = 128
NWIN = EPW // K
REM = EPW - NWIN * K
TPS = NP // NS

def _deg_body(dst_hbm, out_hbm, idx_v, idxr_v, ones_v, zb_v, acc_sh):
    c = lax.axis_index("c")
    s = lax.axis_index("s")
    w = s * NC + c

    def fill_ones(i, carry):
        ones_v[pl.ds(i * 16, 16)] = jnp.ones((16,), jnp.float32)
        return carry

    lax.fori_loop(0, K // 16, fill_ones, 0)

    def fill_zeros(i, carry):
        zb_v[pl.ds(i * 16, 16)] = jnp.zeros((16,), jnp.float32)
        return carry

    lax.fori_loop(0, TPS // 16, fill_zeros, 0)
    pltpu.sync_copy(zb_v, acc_sh.at[pl.ds(s * TPS, TPS)])
    plsc.subcore_barrier()

    base0 = w * EPW

    def win(j, carry):
        pltpu.sync_copy(dst_hbm.at[pl.ds(base0 + j * K, K)], idx_v)
        pltpu.sync_copy(ones_v, acc_sh.at[idx_v], add=True)
        return carry

    lax.fori_loop(0, NWIN, win, 0)
    pltpu.sync_copy(dst_hbm.at[pl.ds(base0 + NWIN * K, REM)], idxr_v)
    pltpu.sync_copy(ones_v.at[pl.ds(0, REM)], acc_sh.at[idxr_v], add=True)
    plsc.subcore_barrier()
    pltpu.sync_copy(acc_sh.at[pl.ds(s * TPS, TPS)],
                    out_hbm.at[pl.ds(c * NP + s * TPS, TPS)])


@functools.lru_cache(maxsize=None)
def _deg_kernel():
    mesh = plsc.VectorSubcoreMesh(core_axis_name="c", subcore_axis_name="s")
    return pl.kernel(
        _deg_body,
        out_type=jax.ShapeDtypeStruct((NC * NP,), jnp.float32),
        mesh=mesh,
        scratch_types=[
            pltpu.VMEM((K,), jnp.int32),
            pltpu.VMEM((REM,), jnp.int32),
            pltpu.VMEM((K,), jnp.float32),
            pltpu.VMEM((TPS,), jnp.float32),
            pltpu.VMEM_SHARED((NP,), jnp.float32),
        ],
    )


def _deg_call(dst):
    return _deg_kernel()(dst)


def _agg_body(src_hbm, dst_hbm, y_hbm, out_hbm,
              si_v, di_v, sir_v, dir_v, rows_v, rowsr_v, zb_v, acc_sh, sem):
    c = lax.axis_index("c")
    s = lax.axis_index("s")
    w = s * NC + c

    def fill_zeros(i, carry):
        zb_v[i // 8, pl.ds((i % 8) * 16, 16)] = jnp.zeros((16,), jnp.float32)
        return carry

    lax.fori_loop(0, (128 * D) // 16, fill_zeros, 0)

    def zcp(t, carry):
        pltpu.sync_copy(zb_v, acc_sh.at[pl.ds(s * TPS + t * 128, 128)])
        return carry

    lax.fori_loop(0, TPS // 128, zcp, 0)
    plsc.subcore_barrier()

    base0 = w * EPW

    def win(j, carry):
        base = base0 + j * K
        pltpu.sync_copy(src_hbm.at[pl.ds(base, K)], si_v)
        pltpu.sync_copy(dst_hbm.at[pl.ds(base, K)], di_v)
        pltpu.async_copy(y_hbm.at[si_v], rows_v, sem).wait()
        pltpu.sync_copy(rows_v, acc_sh.at[di_v], add=True)
        return carry

    lax.fori_loop(0, NWIN, win, 0)

    base = base0 + NWIN * K
    pltpu.sync_copy(src_hbm.at[pl.ds(base, REM)], sir_v)
    pltpu.sync_copy(dst_hbm.at[pl.ds(base, REM)], dir_v)
    pltpu.async_copy(y_hbm.at[sir_v], rowsr_v, sem).wait()
    pltpu.sync_copy(rowsr_v, acc_sh.at[dir_v], add=True)

    plsc.subcore_barrier()
    pltpu.sync_copy(acc_sh.at[pl.ds(s * TPS, TPS)],
                    out_hbm.at[pl.ds(c * NP + s * TPS, TPS)])


@functools.lru_cache(maxsize=None)
def _agg_kernel():
    mesh = plsc.VectorSubcoreMesh(core_axis_name="c", subcore_axis_name="s")
    return pl.kernel(
        _agg_body,
        out_type=jax.ShapeDtypeStruct((NC * NP, D), jnp.float32),
        mesh=mesh,
        scratch_types=[
            pltpu.VMEM((K,), jnp.int32),
            pltpu.VMEM((K,), jnp.int32),
            pltpu.VMEM((REM,), jnp.int32),
            pltpu.VMEM((REM,), jnp.int32),
            pltpu.VMEM((K, D), jnp.float32),
            pltpu.VMEM((REM, D), jnp.float32),
            pltpu.VMEM((128, D), jnp.float32),
            pltpu.VMEM_SHARED((NP, D), jnp.float32),
            pltpu.SemaphoreType.DMA,
        ],
    )


def _agg_call(src, dst, y):
    return _agg_kernel()(src, dst, y)

R = 2000
GRID = N // R


def _tc_first_body(x_ref, w_ref, dis_ref, y_ref):
    y_ref[...] = dis_ref[...] * jnp.dot(
        x_ref[...], w_ref[...], preferred_element_type=jnp.float32)


def _tc_mid_body(a0_ref, a1_ref, yp_ref, dis_ref, b_ref, w_ref, y_ref):
    dis = dis_ref[...]
    h = jnp.tanh(dis * (a0_ref[0] + a1_ref[0] + yp_ref[...]) + b_ref[...])
    y_ref[...] = dis * jnp.dot(h, w_ref[...],
                               preferred_element_type=jnp.float32)


def _tc_final_body(a0_ref, a1_ref, yp_ref, dis_ref, b_ref, wl_ref, bl_ref,
                   batch_ref, out_ref):
    i = pl.program_id(0)
    dis = dis_ref[...]
    h = jnp.tanh(dis * (a0_ref[0] + a1_ref[0] + yp_ref[...]) + b_ref[...])
    t = jnp.tanh(jnp.dot(h, wl_ref[...],
                         preferred_element_type=jnp.float32) + bl_ref[...])
    lane = lax.broadcasted_iota(jnp.int32, (R, D), 1)
    t = jnp.where(lane == 6, 1.0, t)
    oneh = (batch_ref[...] == lane).astype(jnp.float32)
    acc = lax.dot_general(oneh, t, (((0,), (0,)), ((), ())),
                          preferred_element_type=jnp.float32)

    @pl.when(i == 0)
    def _init():
        out_ref[...] = acc

    @pl.when(i > 0)
    def _accum():
        out_ref[...] += acc


_row_spec = pl.BlockSpec((R, D), lambda i: (i, 0))
_w_spec = pl.BlockSpec((D, D), lambda i: (0, 0))
_b_spec = pl.BlockSpec((1, D), lambda i: (0, 0))
_a0_spec = pl.BlockSpec((1, R, D), lambda i: (0, i, 0))
_a1_spec = pl.BlockSpec((1, R, D), lambda i: (1, i, 0))

_tc_first = pl.pallas_call(
    _tc_first_body, grid=(GRID,),
    in_specs=[_row_spec, _w_spec, _row_spec],
    out_specs=_row_spec,
    out_shape=jax.ShapeDtypeStruct((N, D), jnp.float32),
)

_tc_mid = pl.pallas_call(
    _tc_mid_body, grid=(GRID,),
    in_specs=[_a0_spec, _a1_spec, _row_spec, _row_spec, _b_spec, _w_spec],
    out_specs=_row_spec,
    out_shape=jax.ShapeDtypeStruct((N, D), jnp.float32),
)

_tc_final = pl.pallas_call(
    _tc_final_body, grid=(GRID,),
    in_specs=[_a0_spec, _a1_spec, _row_spec, _row_spec, _b_spec, _w_spec,
              _b_spec, _row_spec],
    out_specs=pl.BlockSpec((D, D), lambda i: (0, 0)),
    out_shape=jax.ShapeDtypeStruct((D, D), jnp.float32),
)


def kernel(x, edge_index, batch, W0, b0, W1, b1, W2, b2, Wl, bl):
    src = edge_index[0]
    dst = edge_index[1]

    degp = _deg_call(dst)
    deg = 1.0 + degp[:N] + degp[NP:NP + N]
    dis = lax.rsqrt(deg)
    dis_b = jnp.broadcast_to(dis[:, None], (N, D))

    y0 = _tc_first(x, W0, dis_b)
    a0 = _agg_call(src, dst, y0).reshape(NC, NP, D)
    y1 = _tc_mid(a0, a0, y0, dis_b, b0.reshape(1, D), W1)
    a1 = _agg_call(src, dst, y1).reshape(NC, NP, D)
    y2 = _tc_mid(a1, a1, y1, dis_b, b1.reshape(1, D), W2)
    a2 = _agg_call(src, dst, y2).reshape(NC, NP, D)

    Wlp = jnp.pad(Wl, ((0, 0), (0, D - 6)))
    blp = jnp.pad(bl, (0, D - 6)).reshape(1, D)
    batch_b = jnp.broadcast_to(batch[:, None], (N, D))
    P = _tc_final(a2, a2, y2, dis_b, b2.reshape(1, D), Wlp, blp, batch_b)

    sums = P[:G, :6]
    counts = P[:G, 6]
    return sums / jnp.clip(counts, 1.0)[:, None]


# speedup vs baseline: 18.3362x; 1.2797x over previous
import functools

import jax
import jax.numpy as jnp
from jax import lax
from jax.experimental import pallas as pl
from jax.experimental.pallas import tpu as pltpu
from jax.experimental.pallas import tpu_sc as plsc

N = 10000
E = 320000
D = 128
G = 64
NP = 10240
NC = 2
NS = 16
NWK = NC * NS
K = 128
ER = E // K
NPAIR = ER // 2
TPS = NP // NS
ACCR = 10112
TPSA = ACCR // NS

def _deg_body(dst_hbm, out_hbm, di_v, ones_v, zb_v, acc_sh, sem0, sem1):
    c = lax.axis_index("c")
    s = lax.axis_index("s")
    w = s * NC + c

    def fill_ones(i, carry):
        ones_v[pl.ds(i * 16, 16)] = jnp.ones((16,), jnp.float32)
        return carry

    lax.fori_loop(0, K // 16, fill_ones, 0)

    def fill_zeros(i, carry):
        zb_v[pl.ds(i * 16, 16)] = jnp.zeros((16,), jnp.float32)
        return carry

    lax.fori_loop(0, TPS // 16, fill_zeros, 0)
    pltpu.sync_copy(zb_v, acc_sh.at[pl.ds(s * TPS, TPS)])
    plsc.subcore_barrier()

    plo = (w * NPAIR) // NWK
    phi = ((w + 1) * NPAIR) // NWK

    def pair(t, carry):
        row0 = (plo + t) * 2
        pltpu.sync_copy(dst_hbm.at[pl.ds(row0, 2)], di_v)
        c0 = pltpu.async_copy(ones_v, acc_sh.at[di_v.at[0]], sem0, add=True)
        c1 = pltpu.async_copy(ones_v, acc_sh.at[di_v.at[1]], sem1, add=True)
        c0.wait()
        c1.wait()
        return carry

    lax.fori_loop(0, phi - plo, pair, 0)
    plsc.subcore_barrier()
    pltpu.sync_copy(acc_sh.at[pl.ds(s * TPS, TPS)],
                    out_hbm.at[pl.ds(c * NP + s * TPS, TPS)])


@functools.lru_cache(maxsize=None)
def _deg_kernel():
    mesh = plsc.VectorSubcoreMesh(core_axis_name="c", subcore_axis_name="s")
    return pl.kernel(
        _deg_body,
        out_type=jax.ShapeDtypeStruct((NC * NP,), jnp.float32),
        mesh=mesh,
        scratch_types=[
            pltpu.VMEM((2, K), jnp.int32),
            pltpu.VMEM((K,), jnp.float32),
            pltpu.VMEM((TPS,), jnp.float32),
            pltpu.VMEM_SHARED((NP,), jnp.float32),
            pltpu.SemaphoreType.DMA,
            pltpu.SemaphoreType.DMA,
        ],
    )


def _deg_call(dst):
    return _deg_kernel()(dst)


def _agg_body(src_hbm, dst_hbm, y_hbm, out_hbm,
              si_v, di_v, r0_v, r1_v, zb_v, acc_sh, sem0, sem1):
    c = lax.axis_index("c")
    s = lax.axis_index("s")
    w = s * NC + c

    def fill_zeros(i, carry):
        zb_v[i // 8, pl.ds((i % 8) * 16, 16)] = jnp.zeros((16,), jnp.float32)
        return carry

    lax.fori_loop(0, (128 * D) // 16, fill_zeros, 0)

    def zcp(t, carry):
        pltpu.sync_copy(zb_v, acc_sh.at[pl.ds(s * TPSA + t * 128, 128)])
        return carry

    lax.fori_loop(0, TPSA // 128, zcp, 0)
    pltpu.sync_copy(zb_v.at[pl.ds(0, TPSA % 128)],
                    acc_sh.at[pl.ds(s * TPSA + (TPSA // 128) * 128,
                                    TPSA % 128)])
    plsc.subcore_barrier()

    plo = (w * NPAIR) // NWK
    phi = ((w + 1) * NPAIR) // NWK

    def pair(t, carry):
        row0 = (plo + t) * 2
        pltpu.sync_copy(src_hbm.at[pl.ds(row0, 2)], si_v)
        pltpu.sync_copy(dst_hbm.at[pl.ds(row0, 2)], di_v)
        g0 = pltpu.async_copy(y_hbm.at[si_v.at[0]], r0_v, sem0)
        g1 = pltpu.async_copy(y_hbm.at[si_v.at[1]], r1_v, sem1)
        g0.wait()
        pltpu.sync_copy(r0_v, acc_sh.at[di_v.at[0]], add=True)
        g1.wait()
        pltpu.sync_copy(r1_v, acc_sh.at[di_v.at[1]], add=True)
        return carry

    lax.fori_loop(0, phi - plo, pair, 0)
    plsc.subcore_barrier()
    pltpu.sync_copy(acc_sh.at[pl.ds(s * TPSA, TPSA)],
                    out_hbm.at[pl.ds(c * ACCR + s * TPSA, TPSA)])


@functools.lru_cache(maxsize=None)
def _agg_kernel():
    mesh = plsc.VectorSubcoreMesh(core_axis_name="c", subcore_axis_name="s")
    return pl.kernel(
        _agg_body,
        out_type=jax.ShapeDtypeStruct((NC * ACCR, D), jnp.float32),
        mesh=mesh,
        scratch_types=[
            pltpu.VMEM((2, K), jnp.int32),
            pltpu.VMEM((2, K), jnp.int32),
            pltpu.VMEM((K, D), jnp.float32),
            pltpu.VMEM((K, D), jnp.float32),
            pltpu.VMEM((128, D), jnp.float32),
            pltpu.VMEM_SHARED((ACCR, D), jnp.float32),
            pltpu.SemaphoreType.DMA,
            pltpu.SemaphoreType.DMA,
        ],
    )


def _agg_call(src, dst, y):
    return _agg_kernel()(src, dst, y)

R = 2000
GRID = N // R


def _tc_first_body(x_ref, w_ref, dis_ref, y_ref):
    y_ref[...] = dis_ref[...] * jnp.dot(
        x_ref[...], w_ref[...], preferred_element_type=jnp.float32)


def _tc_mid_body(a0_ref, a1_ref, yp_ref, dis_ref, b_ref, w_ref, y_ref):
    dis = dis_ref[...]
    h = jnp.tanh(dis * (a0_ref[0] + a1_ref[0] + yp_ref[...]) + b_ref[...])
    y_ref[...] = dis * jnp.dot(h, w_ref[...],
                               preferred_element_type=jnp.float32)


def _tc_final_body(a0_ref, a1_ref, yp_ref, dis_ref, b_ref, wl_ref, bl_ref,
                   batch_ref, out_ref):
    i = pl.program_id(0)
    dis = dis_ref[...]
    h = jnp.tanh(dis * (a0_ref[0] + a1_ref[0] + yp_ref[...]) + b_ref[...])
    t = jnp.tanh(jnp.dot(h, wl_ref[...],
                         preferred_element_type=jnp.float32) + bl_ref[...])
    lane = lax.broadcasted_iota(jnp.int32, (R, D), 1)
    t = jnp.where(lane == 6, 1.0, t)
    oneh = (batch_ref[...] == lane).astype(jnp.float32)
    acc = lax.dot_general(oneh, t, (((0,), (0,)), ((), ())),
                          preferred_element_type=jnp.float32)

    @pl.when(i == 0)
    def _init():
        out_ref[...] = acc

    @pl.when(i > 0)
    def _accum():
        out_ref[...] += acc


_row_spec = pl.BlockSpec((R, D), lambda i: (i, 0))
_w_spec = pl.BlockSpec((D, D), lambda i: (0, 0))
_b_spec = pl.BlockSpec((1, D), lambda i: (0, 0))
_a0_spec = pl.BlockSpec((1, R, D), lambda i: (0, i, 0))
_a1_spec = pl.BlockSpec((1, R, D), lambda i: (1, i, 0))

_tc_first = pl.pallas_call(
    _tc_first_body, grid=(GRID,),
    in_specs=[_row_spec, _w_spec, _row_spec],
    out_specs=_row_spec,
    out_shape=jax.ShapeDtypeStruct((N, D), jnp.float32),
)

_tc_mid = pl.pallas_call(
    _tc_mid_body, grid=(GRID,),
    in_specs=[_a0_spec, _a1_spec, _row_spec, _row_spec, _b_spec, _w_spec],
    out_specs=_row_spec,
    out_shape=jax.ShapeDtypeStruct((N, D), jnp.float32),
)

_tc_final = pl.pallas_call(
    _tc_final_body, grid=(GRID,),
    in_specs=[_a0_spec, _a1_spec, _row_spec, _row_spec, _b_spec, _w_spec,
              _b_spec, _row_spec],
    out_specs=pl.BlockSpec((D, D), lambda i: (0, 0)),
    out_shape=jax.ShapeDtypeStruct((D, D), jnp.float32),
)


def kernel(x, edge_index, batch, W0, b0, W1, b1, W2, b2, Wl, bl):
    src = edge_index[0].reshape(ER, K)
    dst = edge_index[1].reshape(ER, K)

    degp = _deg_call(dst)
    deg = 1.0 + degp[:N] + degp[NP:NP + N]
    dis = lax.rsqrt(deg)
    dis_b = jnp.broadcast_to(dis[:, None], (N, D))

    y0 = _tc_first(x, W0, dis_b)
    a0 = _agg_call(src, dst, y0).reshape(NC, ACCR, D)
    y1 = _tc_mid(a0, a0, y0, dis_b, b0.reshape(1, D), W1)
    a1 = _agg_call(src, dst, y1).reshape(NC, ACCR, D)
    y2 = _tc_mid(a1, a1, y1, dis_b, b1.reshape(1, D), W2)
    a2 = _agg_call(src, dst, y2).reshape(NC, ACCR, D)

    Wlp = jnp.pad(Wl, ((0, 0), (0, D - 6)))
    blp = jnp.pad(bl, (0, D - 6)).reshape(1, D)
    batch_b = jnp.broadcast_to(batch[:, None], (N, D))
    P = _tc_final(a2, a2, y2, dis_b, b2.reshape(1, D), Wlp, blp, batch_b)

    sums = P[:G, :6]
    counts = P[:G, 6]
    return sums / jnp.clip(counts, 1.0)[:, None]


# speedup vs baseline: 21.0376x; 1.1473x over previous
import functools

import jax
import jax.numpy as jnp
from jax import lax
from jax.experimental import pallas as pl
from jax.experimental.pallas import tpu as pltpu
from jax.experimental.pallas import tpu_sc as plsc

N = 10000
E = 320000
D = 128
G = 64
NP = 10240
NC = 2
NS = 16
NWK = NC * NS
K = 128
ER = E // K
NPAIR = ER // 2
ERPAD = 2504
TPS = NP // NS
ACCR = 10112
TPSA = ACCR // NS

def _deg_body(dst_hbm, out_hbm, di_v, ones_v, zb_v, acc_sh, sem0, sem1):
    c = lax.axis_index("c")
    s = lax.axis_index("s")
    w = s * NC + c

    def fill_ones(i, carry):
        ones_v[pl.ds(i * 16, 16)] = jnp.ones((16,), jnp.float32)
        return carry

    lax.fori_loop(0, K // 16, fill_ones, 0)

    def fill_zeros(i, carry):
        zb_v[pl.ds(i * 16, 16)] = jnp.zeros((16,), jnp.float32)
        return carry

    lax.fori_loop(0, TPS // 16, fill_zeros, 0)
    pltpu.sync_copy(zb_v, acc_sh.at[pl.ds(s * TPS, TPS)])
    plsc.subcore_barrier()

    rlo = pl.multiple_of(((ER * w) // (NWK * 8)) * 8, 8)
    rhi = jnp.where(w == NWK - 1, ER, ((ER * (w + 1)) // (NWK * 8)) * 8)
    n = (rhi - rlo) // 2

    def chunk(cc, carry):
        base = pl.multiple_of(rlo + cc * 8, 8)
        pltpu.sync_copy(dst_hbm.at[pl.ds(base, 8)], di_v)
        npp = jnp.minimum(4, n - cc * 4)

        def pair(t, carry2):
            c0 = pltpu.async_copy(ones_v, acc_sh.at[di_v.at[2 * t]], sem0,
                                  add=True)
            c1 = pltpu.async_copy(ones_v, acc_sh.at[di_v.at[2 * t + 1]],
                                  sem1, add=True)
            c0.wait()
            c1.wait()
            return carry2

        lax.fori_loop(0, npp, pair, 0)
        return carry

    lax.fori_loop(0, (n + 3) // 4, chunk, 0)
    plsc.subcore_barrier()
    pltpu.sync_copy(acc_sh.at[pl.ds(s * TPS, TPS)],
                    out_hbm.at[pl.ds(c * NP + s * TPS, TPS)])


@functools.lru_cache(maxsize=None)
def _deg_kernel():
    mesh = plsc.VectorSubcoreMesh(core_axis_name="c", subcore_axis_name="s")
    return pl.kernel(
        _deg_body,
        out_type=jax.ShapeDtypeStruct((NC * NP,), jnp.float32),
        mesh=mesh,
        scratch_types=[
            pltpu.VMEM((8, K), jnp.int32),
            pltpu.VMEM((K,), jnp.float32),
            pltpu.VMEM((TPS,), jnp.float32),
            pltpu.VMEM_SHARED((NP,), jnp.float32),
            pltpu.SemaphoreType.DMA,
            pltpu.SemaphoreType.DMA,
        ],
    )


def _deg_call(dst):
    return _deg_kernel()(dst)


def _agg_body(src_hbm, dst_hbm, y_hbm, out_hbm,
              si_v, di_v, r0_v, r1_v, zb_v, acc_sh, sem0, sem1):
    c = lax.axis_index("c")
    s = lax.axis_index("s")
    w = s * NC + c

    def fill_zeros(i, carry):
        zb_v[i // 8, pl.ds((i % 8) * 16, 16)] = jnp.zeros((16,), jnp.float32)
        return carry

    lax.fori_loop(0, (8 * D) // 16, fill_zeros, 0)

    def zcp(t, carry):
        pltpu.sync_copy(zb_v, acc_sh.at[pl.ds(s * TPSA + t * 8, 8)])
        return carry

    lax.fori_loop(0, TPSA // 8, zcp, 0)
    plsc.subcore_barrier()

    rlo = pl.multiple_of(((ER * w) // (NWK * 8)) * 8, 8)
    rhi = jnp.where(w == NWK - 1, ER, ((ER * (w + 1)) // (NWK * 8)) * 8)
    n = (rhi - rlo) // 2

    def chunk(cc, carry):
        base = pl.multiple_of(rlo + cc * 8, 8)
        pltpu.sync_copy(src_hbm.at[pl.ds(base, 8)], si_v)
        pltpu.sync_copy(dst_hbm.at[pl.ds(base, 8)], di_v)
        npp = jnp.minimum(4, n - cc * 4)

        def pair(t, carry2):
            g0 = pltpu.async_copy(y_hbm.at[si_v.at[2 * t]], r0_v, sem0)
            g1 = pltpu.async_copy(y_hbm.at[si_v.at[2 * t + 1]], r1_v, sem1)
            g0.wait()
            pltpu.sync_copy(r0_v, acc_sh.at[di_v.at[2 * t]], add=True)
            g1.wait()
            pltpu.sync_copy(r1_v, acc_sh.at[di_v.at[2 * t + 1]], add=True)
            return carry2

        lax.fori_loop(0, npp, pair, 0)
        return carry

    lax.fori_loop(0, (n + 3) // 4, chunk, 0)
    plsc.subcore_barrier()
    pltpu.sync_copy(acc_sh.at[pl.ds(s * TPSA, TPSA)],
                    out_hbm.at[pl.ds(c * ACCR + s * TPSA, TPSA)])


@functools.lru_cache(maxsize=None)
def _agg_kernel():
    mesh = plsc.VectorSubcoreMesh(core_axis_name="c", subcore_axis_name="s")
    return pl.kernel(
        _agg_body,
        out_type=jax.ShapeDtypeStruct((NC * ACCR, D), jnp.float32),
        mesh=mesh,
        scratch_types=[
            pltpu.VMEM((8, K), jnp.int32),
            pltpu.VMEM((8, K), jnp.int32),
            pltpu.VMEM((K, D), jnp.float32),
            pltpu.VMEM((K, D), jnp.float32),
            pltpu.VMEM((8, D), jnp.float32),
            pltpu.VMEM_SHARED((ACCR, D), jnp.float32),
            pltpu.SemaphoreType.DMA,
            pltpu.SemaphoreType.DMA,
        ],
    )


def _agg_call(src, dst, y):
    return _agg_kernel()(src, dst, y)

R = 2000
GRID = N // R


def _tc_first_body(x_ref, w_ref, dis_ref, y_ref):
    y_ref[...] = dis_ref[...] * jnp.dot(
        x_ref[...], w_ref[...], preferred_element_type=jnp.float32)


def _tc_mid_body(a0_ref, a1_ref, yp_ref, dis_ref, b_ref, w_ref, y_ref):
    dis = dis_ref[...]
    h = jnp.tanh(dis * (a0_ref[0] + a1_ref[0] + yp_ref[...]) + b_ref[...])
    y_ref[...] = dis * jnp.dot(h, w_ref[...],
                               preferred_element_type=jnp.float32)


def _tc_final_body(a0_ref, a1_ref, yp_ref, dis_ref, b_ref, wl_ref, bl_ref,
                   batch_ref, out_ref):
    i = pl.program_id(0)
    dis = dis_ref[...]
    h = jnp.tanh(dis * (a0_ref[0] + a1_ref[0] + yp_ref[...]) + b_ref[...])
    t = jnp.tanh(jnp.dot(h, wl_ref[...],
                         preferred_element_type=jnp.float32) + bl_ref[...])
    lane = lax.broadcasted_iota(jnp.int32, (R, D), 1)
    t = jnp.where(lane == 6, 1.0, t)
    oneh = (batch_ref[...] == lane).astype(jnp.float32)
    acc = lax.dot_general(oneh, t, (((0,), (0,)), ((), ())),
                          preferred_element_type=jnp.float32)

    @pl.when(i == 0)
    def _init():
        out_ref[...] = acc

    @pl.when(i > 0)
    def _accum():
        out_ref[...] += acc


_row_spec = pl.BlockSpec((R, D), lambda i: (i, 0))
_w_spec = pl.BlockSpec((D, D), lambda i: (0, 0))
_b_spec = pl.BlockSpec((1, D), lambda i: (0, 0))
_a0_spec = pl.BlockSpec((1, R, D), lambda i: (0, i, 0))
_a1_spec = pl.BlockSpec((1, R, D), lambda i: (1, i, 0))

_tc_first = pl.pallas_call(
    _tc_first_body, grid=(GRID,),
    in_specs=[_row_spec, _w_spec, _row_spec],
    out_specs=_row_spec,
    out_shape=jax.ShapeDtypeStruct((N, D), jnp.float32),
)

_tc_mid = pl.pallas_call(
    _tc_mid_body, grid=(GRID,),
    in_specs=[_a0_spec, _a1_spec, _row_spec, _row_spec, _b_spec, _w_spec],
    out_specs=_row_spec,
    out_shape=jax.ShapeDtypeStruct((N, D), jnp.float32),
)

_tc_final = pl.pallas_call(
    _tc_final_body, grid=(GRID,),
    in_specs=[_a0_spec, _a1_spec, _row_spec, _row_spec, _b_spec, _w_spec,
              _b_spec, _row_spec],
    out_specs=pl.BlockSpec((D, D), lambda i: (0, 0)),
    out_shape=jax.ShapeDtypeStruct((D, D), jnp.float32),
)


def kernel(x, edge_index, batch, W0, b0, W1, b1, W2, b2, Wl, bl):
    pad = ((0, ERPAD - ER), (0, 0))
    src = jnp.pad(edge_index[0].reshape(ER, K), pad)
    dst = jnp.pad(edge_index[1].reshape(ER, K), pad)

    degp = _deg_call(dst)
    deg = 1.0 + degp[:N] + degp[NP:NP + N]
    dis = lax.rsqrt(deg)
    dis_b = jnp.broadcast_to(dis[:, None], (N, D))

    y0 = _tc_first(x, W0, dis_b)
    a0 = _agg_call(src, dst, y0).reshape(NC, ACCR, D)
    y1 = _tc_mid(a0, a0, y0, dis_b, b0.reshape(1, D), W1)
    a1 = _agg_call(src, dst, y1).reshape(NC, ACCR, D)
    y2 = _tc_mid(a1, a1, y1, dis_b, b1.reshape(1, D), W2)
    a2 = _agg_call(src, dst, y2).reshape(NC, ACCR, D)

    Wlp = jnp.pad(Wl, ((0, 0), (0, D - 6)))
    blp = jnp.pad(bl, (0, D - 6)).reshape(1, D)
    batch_b = jnp.broadcast_to(batch[:, None], (N, D))
    P = _tc_final(a2, a2, y2, dis_b, b2.reshape(1, D), Wlp, blp, batch_b)

    sums = P[:G, :6]
    counts = P[:G, 6]
    return sums / jnp.clip(counts, 1.0)[:, None]


# speedup vs baseline: 25.0527x; 1.1909x over previous
import functools

import jax
import jax.numpy as jnp
from jax import lax
from jax.experimental import pallas as pl
from jax.experimental.pallas import tpu as pltpu
from jax.experimental.pallas import tpu_sc as plsc

N = 10000
E = 320000
D = 128
G = 64
NP = 10240
NC = 2
NS = 16
NWK = NC * NS
K = 128
ER = E // K
NPAIR = ER // 2
ERPAD = 2504
TPS = NP // NS
ACCR = 10112
TPSA = ACCR // NS

def _deg_body(dst_hbm, out_hbm, di_v, ones_v, zb_v, acc_sh, sem0, sem1):
    c = lax.axis_index("c")
    s = lax.axis_index("s")
    w = s * NC + c

    def fill_ones(i, carry):
        ones_v[pl.ds(i * 16, 16)] = jnp.ones((16,), jnp.float32)
        return carry

    lax.fori_loop(0, K // 16, fill_ones, 0)

    def fill_zeros(i, carry):
        zb_v[pl.ds(i * 16, 16)] = jnp.zeros((16,), jnp.float32)
        return carry

    lax.fori_loop(0, TPS // 16, fill_zeros, 0)
    pltpu.sync_copy(zb_v, acc_sh.at[pl.ds(s * TPS, TPS)])
    plsc.subcore_barrier()

    rlo = pl.multiple_of(((ER * w) // (NWK * 8)) * 8, 8)
    rhi = jnp.where(w == NWK - 1, ER, ((ER * (w + 1)) // (NWK * 8)) * 8)
    n = (rhi - rlo) // 2

    def chunk(cc, carry):
        base = pl.multiple_of(rlo + cc * 8, 8)
        pltpu.sync_copy(dst_hbm.at[pl.ds(base, 8)], di_v)
        npp = jnp.minimum(4, n - cc * 4)

        def pair(t, carry2):
            c0 = pltpu.async_copy(ones_v, acc_sh.at[di_v.at[2 * t]], sem0,
                                  add=True)
            c1 = pltpu.async_copy(ones_v, acc_sh.at[di_v.at[2 * t + 1]],
                                  sem1, add=True)
            c0.wait()
            c1.wait()
            return carry2

        lax.fori_loop(0, npp, pair, 0)
        return carry

    lax.fori_loop(0, (n + 3) // 4, chunk, 0)
    plsc.subcore_barrier()
    pltpu.sync_copy(acc_sh.at[pl.ds(s * TPS, TPS)],
                    out_hbm.at[pl.ds(c * NP + s * TPS, TPS)])


@functools.lru_cache(maxsize=None)
def _deg_kernel():
    mesh = plsc.VectorSubcoreMesh(core_axis_name="c", subcore_axis_name="s")
    return pl.kernel(
        _deg_body,
        out_type=jax.ShapeDtypeStruct((NC * NP,), jnp.float32),
        mesh=mesh,
        scratch_types=[
            pltpu.VMEM((8, K), jnp.int32),
            pltpu.VMEM((K,), jnp.float32),
            pltpu.VMEM((TPS,), jnp.float32),
            pltpu.VMEM_SHARED((NP,), jnp.float32),
            pltpu.SemaphoreType.DMA,
            pltpu.SemaphoreType.DMA,
        ],
    )


def _deg_call(dst):
    return _deg_kernel()(dst)


def _agg_body(src_hbm, dst_hbm, y_hbm, out_hbm,
              si_v, di_v, r0_v, r1_v, zb_v, acc_sh, sem0, sem1):
    c = lax.axis_index("c")
    s = lax.axis_index("s")
    w = s * NC + c

    def fill_zeros(i, carry):
        zb_v[i // 8, pl.ds((i % 8) * 16, 16)] = jnp.zeros((16,), jnp.float32)
        return carry

    lax.fori_loop(0, (8 * D) // 16, fill_zeros, 0)

    def zcp(t, carry):
        pltpu.sync_copy(zb_v, acc_sh.at[pl.ds(s * TPSA + t * 8, 8)])
        return carry

    lax.fori_loop(0, TPSA // 8, zcp, 0)
    plsc.subcore_barrier()

    rlo = pl.multiple_of(((ER * w) // (NWK * 8)) * 8, 8)
    rhi = jnp.where(w == NWK - 1, ER, ((ER * (w + 1)) // (NWK * 8)) * 8)
    n = (rhi - rlo) // 2

    def chunk(cc, carry):
        base = pl.multiple_of(rlo + cc * 8, 8)
        pltpu.sync_copy(src_hbm.at[pl.ds(base, 8)], si_v)
        pltpu.sync_copy(dst_hbm.at[pl.ds(base, 8)], di_v)
        npp = jnp.minimum(4, n - cc * 4)
        pltpu.async_copy(y_hbm.at[si_v.at[0]], r0_v, sem0)
        pltpu.async_copy(y_hbm.at[si_v.at[1]], r1_v, sem1)

        def pair(t, carry2):
            pltpu.make_async_copy(y_hbm.at[si_v.at[0]], r0_v, sem0).wait()
            pltpu.sync_copy(r0_v, acc_sh.at[di_v.at[2 * t]], add=True)

            @pl.when(t < npp - 1)
            def _g0next():
                pltpu.async_copy(y_hbm.at[si_v.at[2 * t + 2]], r0_v, sem0)

            pltpu.make_async_copy(y_hbm.at[si_v.at[1]], r1_v, sem1).wait()
            pltpu.sync_copy(r1_v, acc_sh.at[di_v.at[2 * t + 1]], add=True)

            @pl.when(t < npp - 1)
            def _g1next():
                pltpu.async_copy(y_hbm.at[si_v.at[2 * t + 3]], r1_v, sem1)

            return carry2

        lax.fori_loop(0, npp, pair, 0)
        return carry

    lax.fori_loop(0, (n + 3) // 4, chunk, 0)
    plsc.subcore_barrier()
    pltpu.sync_copy(acc_sh.at[pl.ds(s * TPSA, TPSA)],
                    out_hbm.at[pl.ds(c * ACCR + s * TPSA, TPSA)])


@functools.lru_cache(maxsize=None)
def _agg_kernel():
    mesh = plsc.VectorSubcoreMesh(core_axis_name="c", subcore_axis_name="s")
    return pl.kernel(
        _agg_body,
        out_type=jax.ShapeDtypeStruct((NC * ACCR, D), jnp.float32),
        mesh=mesh,
        scratch_types=[
            pltpu.VMEM((8, K), jnp.int32),
            pltpu.VMEM((8, K), jnp.int32),
            pltpu.VMEM((K, D), jnp.float32),
            pltpu.VMEM((K, D), jnp.float32),
            pltpu.VMEM((8, D), jnp.float32),
            pltpu.VMEM_SHARED((ACCR, D), jnp.float32),
            pltpu.SemaphoreType.DMA,
            pltpu.SemaphoreType.DMA,
        ],
    )


def _agg_call(src, dst, y):
    return _agg_kernel()(src, dst, y)

R = 2000
GRID = N // R


def _tc_first_body(x_ref, w_ref, dis_ref, y_ref):
    y_ref[...] = dis_ref[...] * jnp.dot(
        x_ref[...], w_ref[...], preferred_element_type=jnp.float32)


def _tc_mid_body(a0_ref, a1_ref, yp_ref, dis_ref, b_ref, w_ref, y_ref):
    dis = dis_ref[...]
    h = jnp.tanh(dis * (a0_ref[0] + a1_ref[0] + yp_ref[...]) + b_ref[...])
    y_ref[...] = dis * jnp.dot(h, w_ref[...],
                               preferred_element_type=jnp.float32)


def _tc_final_body(a0_ref, a1_ref, yp_ref, dis_ref, b_ref, wl_ref, bl_ref,
                   batch_ref, out_ref):
    i = pl.program_id(0)
    dis = dis_ref[...]
    h = jnp.tanh(dis * (a0_ref[0] + a1_ref[0] + yp_ref[...]) + b_ref[...])
    t = jnp.tanh(jnp.dot(h, wl_ref[...],
                         preferred_element_type=jnp.float32) + bl_ref[...])
    lane = lax.broadcasted_iota(jnp.int32, (R, D), 1)
    t = jnp.where(lane == 6, 1.0, t)
    oneh = (batch_ref[...] == lane).astype(jnp.float32)
    acc = lax.dot_general(oneh, t, (((0,), (0,)), ((), ())),
                          preferred_element_type=jnp.float32)

    @pl.when(i == 0)
    def _init():
        out_ref[...] = acc

    @pl.when(i > 0)
    def _accum():
        out_ref[...] += acc


_row_spec = pl.BlockSpec((R, D), lambda i: (i, 0))
_w_spec = pl.BlockSpec((D, D), lambda i: (0, 0))
_b_spec = pl.BlockSpec((1, D), lambda i: (0, 0))
_a0_spec = pl.BlockSpec((1, R, D), lambda i: (0, i, 0))
_a1_spec = pl.BlockSpec((1, R, D), lambda i: (1, i, 0))

_tc_first = pl.pallas_call(
    _tc_first_body, grid=(GRID,),
    in_specs=[_row_spec, _w_spec, _row_spec],
    out_specs=_row_spec,
    out_shape=jax.ShapeDtypeStruct((N, D), jnp.float32),
)

_tc_mid = pl.pallas_call(
    _tc_mid_body, grid=(GRID,),
    in_specs=[_a0_spec, _a1_spec, _row_spec, _row_spec, _b_spec, _w_spec],
    out_specs=_row_spec,
    out_shape=jax.ShapeDtypeStruct((N, D), jnp.float32),
)

_tc_final = pl.pallas_call(
    _tc_final_body, grid=(GRID,),
    in_specs=[_a0_spec, _a1_spec, _row_spec, _row_spec, _b_spec, _w_spec,
              _b_spec, _row_spec],
    out_specs=pl.BlockSpec((D, D), lambda i: (0, 0)),
    out_shape=jax.ShapeDtypeStruct((D, D), jnp.float32),
)


def kernel(x, edge_index, batch, W0, b0, W1, b1, W2, b2, Wl, bl):
    pad = ((0, ERPAD - ER), (0, 0))
    src = jnp.pad(edge_index[0].reshape(ER, K), pad)
    dst = jnp.pad(edge_index[1].reshape(ER, K), pad)

    degp = _deg_call(dst)
    deg = 1.0 + degp[:N] + degp[NP:NP + N]
    dis = lax.rsqrt(deg)
    dis_b = jnp.broadcast_to(dis[:, None], (N, D))

    y0 = _tc_first(x, W0, dis_b)
    a0 = _agg_call(src, dst, y0).reshape(NC, ACCR, D)
    y1 = _tc_mid(a0, a0, y0, dis_b, b0.reshape(1, D), W1)
    a1 = _agg_call(src, dst, y1).reshape(NC, ACCR, D)
    y2 = _tc_mid(a1, a1, y1, dis_b, b1.reshape(1, D), W2)
    a2 = _agg_call(src, dst, y2).reshape(NC, ACCR, D)

    Wlp = jnp.pad(Wl, ((0, 0), (0, D - 6)))
    blp = jnp.pad(bl, (0, D - 6)).reshape(1, D)
    batch_b = jnp.broadcast_to(batch[:, None], (N, D))
    P = _tc_final(a2, a2, y2, dis_b, b2.reshape(1, D), Wlp, blp, batch_b)

    sums = P[:G, :6]
    counts = P[:G, 6]
    return sums / jnp.clip(counts, 1.0)[:, None]


# speedup vs baseline: 27.9011x; 1.1137x over previous
import functools

import jax
import jax.numpy as jnp
from jax import lax
from jax.experimental import pallas as pl
from jax.experimental.pallas import tpu as pltpu
from jax.experimental.pallas import tpu_sc as plsc

N = 10000
E = 320000
D = 128
G = 64
NP = 10240
NC = 2
NS = 16
NWK = NC * NS
K = 128
ER = E // K
NPAIR = ER // 2
ERPAD = 2528
TPS = NP // NS
ACCR = 10112
TPSA = ACCR // NS

def _deg_body(dst_hbm, out_hbm, di_v, ones_v, zb_v, acc_sh, sem0, sem1):
    c = lax.axis_index("c")
    s = lax.axis_index("s")
    w = s * NC + c

    def fill_ones(i, carry):
        ones_v[pl.ds(i * 16, 16)] = jnp.ones((16,), jnp.float32)
        return carry

    lax.fori_loop(0, K // 16, fill_ones, 0)

    def fill_zeros(i, carry):
        zb_v[pl.ds(i * 16, 16)] = jnp.zeros((16,), jnp.float32)
        return carry

    lax.fori_loop(0, TPS // 16, fill_zeros, 0)
    pltpu.sync_copy(zb_v, acc_sh.at[pl.ds(s * TPS, TPS)])
    plsc.subcore_barrier()

    rlo = pl.multiple_of(((ER * w) // (NWK * 8)) * 8, 8)
    rhi = jnp.where(w == NWK - 1, ER, ((ER * (w + 1)) // (NWK * 8)) * 8)
    n = (rhi - rlo) // 2

    def chunk(cc, carry):
        base = pl.multiple_of(rlo + cc * 8, 8)
        pltpu.sync_copy(dst_hbm.at[pl.ds(base, 8)], di_v)
        npp = jnp.minimum(4, n - cc * 4)

        def pair(t, carry2):
            c0 = pltpu.async_copy(ones_v, acc_sh.at[di_v.at[2 * t]], sem0,
                                  add=True)
            c1 = pltpu.async_copy(ones_v, acc_sh.at[di_v.at[2 * t + 1]],
                                  sem1, add=True)
            c0.wait()
            c1.wait()
            return carry2

        lax.fori_loop(0, npp, pair, 0)
        return carry

    lax.fori_loop(0, (n + 3) // 4, chunk, 0)
    plsc.subcore_barrier()
    pltpu.sync_copy(acc_sh.at[pl.ds(s * TPS, TPS)],
                    out_hbm.at[pl.ds(c * NP + s * TPS, TPS)])


@functools.lru_cache(maxsize=None)
def _deg_kernel():
    mesh = plsc.VectorSubcoreMesh(core_axis_name="c", subcore_axis_name="s")
    return pl.kernel(
        _deg_body,
        out_type=jax.ShapeDtypeStruct((NC * NP,), jnp.float32),
        mesh=mesh,
        scratch_types=[
            pltpu.VMEM((8, K), jnp.int32),
            pltpu.VMEM((K,), jnp.float32),
            pltpu.VMEM((TPS,), jnp.float32),
            pltpu.VMEM_SHARED((NP,), jnp.float32),
            pltpu.SemaphoreType.DMA,
            pltpu.SemaphoreType.DMA,
        ],
    )


def _deg_call(dst):
    return _deg_kernel()(dst)


def _agg_body(src_hbm, dst_hbm, y_hbm, out_hbm,
              si_v, di_v, r0_v, r1_v, zb_v, acc_sh, sem0, sem1):
    c = lax.axis_index("c")
    s = lax.axis_index("s")
    w = s * NC + c

    def fill_zeros(i, carry):
        zb_v[i // 8, pl.ds((i % 8) * 16, 16)] = jnp.zeros((16,), jnp.float32)
        return carry

    lax.fori_loop(0, (8 * D) // 16, fill_zeros, 0)

    def zcp(t, carry):
        pltpu.sync_copy(zb_v, acc_sh.at[pl.ds(s * TPSA + t * 8, 8)])
        return carry

    lax.fori_loop(0, TPSA // 8, zcp, 0)
    plsc.subcore_barrier()

    ulo = (w * 313) // NWK
    uhi = ((w + 1) * 313) // NWK
    rlo = pl.multiple_of(ulo * 8, 8)
    rhi = jnp.minimum(uhi * 8, ER)
    n = (rhi - rlo) // 2

    def chunk(cc, carry):
        base = pl.multiple_of(rlo + cc * 32, 8)
        pltpu.sync_copy(src_hbm.at[pl.ds(base, 32)], si_v)
        pltpu.sync_copy(dst_hbm.at[pl.ds(base, 32)], di_v)
        npp = jnp.minimum(16, n - cc * 16)
        pltpu.async_copy(y_hbm.at[si_v.at[0]], r0_v, sem0)
        pltpu.async_copy(y_hbm.at[si_v.at[1]], r1_v, sem1)

        def pair(t, carry2):
            pltpu.make_async_copy(y_hbm.at[si_v.at[0]], r0_v, sem0).wait()
            pltpu.sync_copy(r0_v, acc_sh.at[di_v.at[2 * t]], add=True)

            @pl.when(t < npp - 1)
            def _g0next():
                pltpu.async_copy(y_hbm.at[si_v.at[2 * t + 2]], r0_v, sem0)

            pltpu.make_async_copy(y_hbm.at[si_v.at[1]], r1_v, sem1).wait()
            pltpu.sync_copy(r1_v, acc_sh.at[di_v.at[2 * t + 1]], add=True)

            @pl.when(t < npp - 1)
            def _g1next():
                pltpu.async_copy(y_hbm.at[si_v.at[2 * t + 3]], r1_v, sem1)

            return carry2

        lax.fori_loop(0, npp, pair, 0)
        return carry

    lax.fori_loop(0, (n + 15) // 16, chunk, 0)
    plsc.subcore_barrier()
    pltpu.sync_copy(acc_sh.at[pl.ds(s * TPSA, TPSA)],
                    out_hbm.at[pl.ds(c * ACCR + s * TPSA, TPSA)])


@functools.lru_cache(maxsize=None)
def _agg_kernel():
    mesh = plsc.VectorSubcoreMesh(core_axis_name="c", subcore_axis_name="s")
    return pl.kernel(
        _agg_body,
        out_type=jax.ShapeDtypeStruct((NC * ACCR, D), jnp.float32),
        mesh=mesh,
        scratch_types=[
            pltpu.VMEM((32, K), jnp.int32),
            pltpu.VMEM((32, K), jnp.int32),
            pltpu.VMEM((K, D), jnp.float32),
            pltpu.VMEM((K, D), jnp.float32),
            pltpu.VMEM((8, D), jnp.float32),
            pltpu.VMEM_SHARED((ACCR, D), jnp.float32),
            pltpu.SemaphoreType.DMA,
            pltpu.SemaphoreType.DMA,
        ],
    )


def _agg_call(src, dst, y):
    return _agg_kernel()(src, dst, y)

R = 2000
GRID = N // R


def _tc_first_body(x_ref, w_ref, dis_ref, y_ref):
    y_ref[...] = dis_ref[...] * jnp.dot(
        x_ref[...], w_ref[...], preferred_element_type=jnp.float32)


def _tc_mid_body(a0_ref, a1_ref, yp_ref, dis_ref, b_ref, w_ref, y_ref):
    dis = dis_ref[...]
    h = jnp.tanh(dis * (a0_ref[0] + a1_ref[0] + yp_ref[...]) + b_ref[...])
    y_ref[...] = dis * jnp.dot(h, w_ref[...],
                               preferred_element_type=jnp.float32)


def _tc_final_body(a0_ref, a1_ref, yp_ref, dis_ref, b_ref, wl_ref, bl_ref,
                   batch_ref, out_ref):
    i = pl.program_id(0)
    dis = dis_ref[...]
    h = jnp.tanh(dis * (a0_ref[0] + a1_ref[0] + yp_ref[...]) + b_ref[...])
    t = jnp.tanh(jnp.dot(h, wl_ref[...],
                         preferred_element_type=jnp.float32) + bl_ref[...])
    lane = lax.broadcasted_iota(jnp.int32, (R, D), 1)
    t = jnp.where(lane == 6, 1.0, t)
    oneh = (batch_ref[...] == lane).astype(jnp.float32)
    acc = lax.dot_general(oneh, t, (((0,), (0,)), ((), ())),
                          preferred_element_type=jnp.float32)

    @pl.when(i == 0)
    def _init():
        out_ref[...] = acc

    @pl.when(i > 0)
    def _accum():
        out_ref[...] += acc


_row_spec = pl.BlockSpec((R, D), lambda i: (i, 0))
_w_spec = pl.BlockSpec((D, D), lambda i: (0, 0))
_b_spec = pl.BlockSpec((1, D), lambda i: (0, 0))
_a0_spec = pl.BlockSpec((1, R, D), lambda i: (0, i, 0))
_a1_spec = pl.BlockSpec((1, R, D), lambda i: (1, i, 0))

_tc_first = pl.pallas_call(
    _tc_first_body, grid=(GRID,),
    in_specs=[_row_spec, _w_spec, _row_spec],
    out_specs=_row_spec,
    out_shape=jax.ShapeDtypeStruct((N, D), jnp.float32),
)

_tc_mid = pl.pallas_call(
    _tc_mid_body, grid=(GRID,),
    in_specs=[_a0_spec, _a1_spec, _row_spec, _row_spec, _b_spec, _w_spec],
    out_specs=_row_spec,
    out_shape=jax.ShapeDtypeStruct((N, D), jnp.float32),
)

_tc_final = pl.pallas_call(
    _tc_final_body, grid=(GRID,),
    in_specs=[_a0_spec, _a1_spec, _row_spec, _row_spec, _b_spec, _w_spec,
              _b_spec, _row_spec],
    out_specs=pl.BlockSpec((D, D), lambda i: (0, 0)),
    out_shape=jax.ShapeDtypeStruct((D, D), jnp.float32),
)


def kernel(x, edge_index, batch, W0, b0, W1, b1, W2, b2, Wl, bl):
    pad = ((0, ERPAD - ER), (0, 0))
    src = jnp.pad(edge_index[0].reshape(ER, K), pad)
    dst = jnp.pad(edge_index[1].reshape(ER, K), pad)

    degp = _deg_call(dst)
    deg = 1.0 + degp[:N] + degp[NP:NP + N]
    dis = lax.rsqrt(deg)
    dis_b = jnp.broadcast_to(dis[:, None], (N, D))

    y0 = _tc_first(x, W0, dis_b)
    a0 = _agg_call(src, dst, y0).reshape(NC, ACCR, D)
    y1 = _tc_mid(a0, a0, y0, dis_b, b0.reshape(1, D), W1)
    a1 = _agg_call(src, dst, y1).reshape(NC, ACCR, D)
    y2 = _tc_mid(a1, a1, y1, dis_b, b1.reshape(1, D), W2)
    a2 = _agg_call(src, dst, y2).reshape(NC, ACCR, D)

    Wlp = jnp.pad(Wl, ((0, 0), (0, D - 6)))
    blp = jnp.pad(bl, (0, D - 6)).reshape(1, D)
    batch_b = jnp.broadcast_to(batch[:, None], (N, D))
    P = _tc_final(a2, a2, y2, dis_b, b2.reshape(1, D), Wlp, blp, batch_b)

    sums = P[:G, :6]
    counts = P[:G, 6]
    return sums / jnp.clip(counts, 1.0)[:, None]


# speedup vs baseline: 28.2980x; 1.0142x over previous
import functools

import jax
import jax.numpy as jnp
from jax import lax
from jax.experimental import pallas as pl
from jax.experimental.pallas import tpu as pltpu
from jax.experimental.pallas import tpu_sc as plsc

N = 10000
E = 320000
D = 128
G = 64
NP = 10240
NC = 2
NS = 16
NWK = NC * NS
K = 128
ER = E // K
NPAIR = ER // 2
ERPAD = 2528
TPS = NP // NS
ACCR = 10112
TPSA = ACCR // NS

def _deg_body(dst_hbm, out_hbm, di_v, ones_v, zb_v, acc_sh, sem0, sem1):
    c = lax.axis_index("c")
    s = lax.axis_index("s")
    w = s * NC + c

    def fill_ones(i, carry):
        ones_v[pl.ds(i * 16, 16)] = jnp.ones((16,), jnp.float32)
        return carry

    lax.fori_loop(0, K // 16, fill_ones, 0)

    def fill_zeros(i, carry):
        zb_v[pl.ds(i * 16, 16)] = jnp.zeros((16,), jnp.float32)
        return carry

    lax.fori_loop(0, TPS // 16, fill_zeros, 0)
    pltpu.sync_copy(zb_v, acc_sh.at[pl.ds(s * TPS, TPS)])
    plsc.subcore_barrier()

    rlo = pl.multiple_of(((ER * w) // (NWK * 8)) * 8, 8)
    rhi = jnp.where(w == NWK - 1, ER, ((ER * (w + 1)) // (NWK * 8)) * 8)
    n = (rhi - rlo) // 2

    def chunk(cc, carry):
        base = pl.multiple_of(rlo + cc * 8, 8)
        pltpu.sync_copy(dst_hbm.at[pl.ds(base, 8)], di_v)
        npp = jnp.minimum(4, n - cc * 4)

        def pair(t, carry2):
            c0 = pltpu.async_copy(ones_v, acc_sh.at[di_v.at[2 * t]], sem0,
                                  add=True)
            c1 = pltpu.async_copy(ones_v, acc_sh.at[di_v.at[2 * t + 1]],
                                  sem1, add=True)
            c0.wait()
            c1.wait()
            return carry2

        lax.fori_loop(0, npp, pair, 0)
        return carry

    lax.fori_loop(0, (n + 3) // 4, chunk, 0)
    plsc.subcore_barrier()
    pltpu.sync_copy(acc_sh.at[pl.ds(s * TPS, TPS)],
                    out_hbm.at[pl.ds(c * NP + s * TPS, TPS)])


@functools.lru_cache(maxsize=None)
def _deg_kernel():
    mesh = plsc.VectorSubcoreMesh(core_axis_name="c", subcore_axis_name="s")
    return pl.kernel(
        _deg_body,
        out_type=jax.ShapeDtypeStruct((NC * NP,), jnp.float32),
        mesh=mesh,
        scratch_types=[
            pltpu.VMEM((8, K), jnp.int32),
            pltpu.VMEM((K,), jnp.float32),
            pltpu.VMEM((TPS,), jnp.float32),
            pltpu.VMEM_SHARED((NP,), jnp.float32),
            pltpu.SemaphoreType.DMA,
            pltpu.SemaphoreType.DMA,
        ],
    )


def _deg_call(dst):
    return _deg_kernel()(dst)


def _agg_body(src_hbm, dst_hbm, y_hbm, out_hbm,
              si_v, di_v, r0_v, r1_v, zb_v, acc_sh, sem0, sem1):
    c = lax.axis_index("c")
    s = lax.axis_index("s")
    w = s * NC + c

    def fill_zeros(i, carry):
        zb_v[i // 8, pl.ds((i % 8) * 16, 16)] = jnp.zeros((16,), jnp.float32)
        return carry

    lax.fori_loop(0, (64 * D) // 16, fill_zeros, 0)

    def zcp(t, carry):
        pltpu.sync_copy(zb_v, acc_sh.at[pl.ds(s * TPSA + t * 64, 64)])
        return carry

    lax.fori_loop(0, 9, zcp, 0)
    pltpu.sync_copy(zb_v.at[pl.ds(0, 56)],
                    acc_sh.at[pl.ds(s * TPSA + 576, 56)])
    plsc.subcore_barrier()

    ulo = (w * 313) // NWK
    uhi = ((w + 1) * 313) // NWK
    rlo = pl.multiple_of(ulo * 8, 8)
    rhi = jnp.minimum(uhi * 8, ER)
    n = (rhi - rlo) // 2

    def chunk(cc, carry):
        base = pl.multiple_of(rlo + cc * 32, 8)
        pltpu.sync_copy(src_hbm.at[pl.ds(base, 32)], si_v)
        pltpu.sync_copy(dst_hbm.at[pl.ds(base, 32)], di_v)
        npp = jnp.minimum(16, n - cc * 16)
        pltpu.async_copy(y_hbm.at[si_v.at[0]], r0_v, sem0)
        pltpu.async_copy(y_hbm.at[si_v.at[1]], r1_v, sem1)

        def pair(t, carry2):
            pltpu.make_async_copy(y_hbm.at[si_v.at[0]], r0_v, sem0).wait()
            pltpu.sync_copy(r0_v, acc_sh.at[di_v.at[2 * t]], add=True)

            @pl.when(t < npp - 1)
            def _g0next():
                pltpu.async_copy(y_hbm.at[si_v.at[2 * t + 2]], r0_v, sem0)

            pltpu.make_async_copy(y_hbm.at[si_v.at[1]], r1_v, sem1).wait()
            pltpu.sync_copy(r1_v, acc_sh.at[di_v.at[2 * t + 1]], add=True)

            @pl.when(t < npp - 1)
            def _g1next():
                pltpu.async_copy(y_hbm.at[si_v.at[2 * t + 3]], r1_v, sem1)

            return carry2

        lax.fori_loop(0, npp, pair, 0)
        return carry

    lax.fori_loop(0, (n + 15) // 16, chunk, 0)
    plsc.subcore_barrier()
    pltpu.sync_copy(acc_sh.at[pl.ds(s * TPSA, TPSA)],
                    out_hbm.at[pl.ds(c * ACCR + s * TPSA, TPSA)])


@functools.lru_cache(maxsize=None)
def _agg_kernel():
    mesh = plsc.VectorSubcoreMesh(core_axis_name="c", subcore_axis_name="s")
    return pl.kernel(
        _agg_body,
        out_type=jax.ShapeDtypeStruct((NC * ACCR, D), jnp.float32),
        mesh=mesh,
        scratch_types=[
            pltpu.VMEM((32, K), jnp.int32),
            pltpu.VMEM((32, K), jnp.int32),
            pltpu.VMEM((K, D), jnp.float32),
            pltpu.VMEM((K, D), jnp.float32),
            pltpu.VMEM((64, D), jnp.float32),
            pltpu.VMEM_SHARED((ACCR, D), jnp.float32),
            pltpu.SemaphoreType.DMA,
            pltpu.SemaphoreType.DMA,
        ],
    )


def _agg_call(src, dst, y):
    return _agg_kernel()(src, dst, y)

R = 2000
GRID = N // R


def _tc_first_body(x_ref, w_ref, dis_ref, y_ref):
    y_ref[...] = dis_ref[...] * jnp.dot(
        x_ref[...], w_ref[...], preferred_element_type=jnp.float32)


def _tc_mid_body(a0_ref, a1_ref, yp_ref, dis_ref, b_ref, w_ref, y_ref):
    dis = dis_ref[...]
    h = jnp.tanh(dis * (a0_ref[0] + a1_ref[0] + yp_ref[...]) + b_ref[...])
    y_ref[...] = dis * jnp.dot(h, w_ref[...],
                               preferred_element_type=jnp.float32)


def _tc_final_body(a0_ref, a1_ref, yp_ref, dis_ref, b_ref, wl_ref, bl_ref,
                   batch_ref, out_ref):
    i = pl.program_id(0)
    dis = dis_ref[...]
    h = jnp.tanh(dis * (a0_ref[0] + a1_ref[0] + yp_ref[...]) + b_ref[...])
    t = jnp.tanh(jnp.dot(h, wl_ref[...],
                         preferred_element_type=jnp.float32) + bl_ref[...])
    lane = lax.broadcasted_iota(jnp.int32, (R, D), 1)
    t = jnp.where(lane == 6, 1.0, t)
    oneh = (batch_ref[...] == lane).astype(jnp.float32)
    acc = lax.dot_general(oneh, t, (((0,), (0,)), ((), ())),
                          preferred_element_type=jnp.float32)

    @pl.when(i == 0)
    def _init():
        out_ref[...] = acc

    @pl.when(i > 0)
    def _accum():
        out_ref[...] += acc


_row_spec = pl.BlockSpec((R, D), lambda i: (i, 0))
_w_spec = pl.BlockSpec((D, D), lambda i: (0, 0))
_b_spec = pl.BlockSpec((1, D), lambda i: (0, 0))
_a0_spec = pl.BlockSpec((1, R, D), lambda i: (0, i, 0))
_a1_spec = pl.BlockSpec((1, R, D), lambda i: (1, i, 0))

_tc_first = pl.pallas_call(
    _tc_first_body, grid=(GRID,),
    in_specs=[_row_spec, _w_spec, _row_spec],
    out_specs=_row_spec,
    out_shape=jax.ShapeDtypeStruct((N, D), jnp.float32),
)

_tc_mid = pl.pallas_call(
    _tc_mid_body, grid=(GRID,),
    in_specs=[_a0_spec, _a1_spec, _row_spec, _row_spec, _b_spec, _w_spec],
    out_specs=_row_spec,
    out_shape=jax.ShapeDtypeStruct((N, D), jnp.float32),
)

_tc_final = pl.pallas_call(
    _tc_final_body, grid=(GRID,),
    in_specs=[_a0_spec, _a1_spec, _row_spec, _row_spec, _b_spec, _w_spec,
              _b_spec, _row_spec],
    out_specs=pl.BlockSpec((D, D), lambda i: (0, 0)),
    out_shape=jax.ShapeDtypeStruct((D, D), jnp.float32),
)


def kernel(x, edge_index, batch, W0, b0, W1, b1, W2, b2, Wl, bl):
    pad = ((0, ERPAD - ER), (0, 0))
    src = jnp.pad(edge_index[0].reshape(ER, K), pad)
    dst = jnp.pad(edge_index[1].reshape(ER, K), pad)

    degp = _deg_call(dst)
    deg = 1.0 + degp[:N] + degp[NP:NP + N]
    dis = lax.rsqrt(deg)
    dis_b = jnp.broadcast_to(dis[:, None], (N, D))

    y0 = _tc_first(x, W0, dis_b)
    a0 = _agg_call(src, dst, y0).reshape(NC, ACCR, D)
    y1 = _tc_mid(a0, a0, y0, dis_b, b0.reshape(1, D), W1)
    a1 = _agg_call(src, dst, y1).reshape(NC, ACCR, D)
    y2 = _tc_mid(a1, a1, y1, dis_b, b1.reshape(1, D), W2)
    a2 = _agg_call(src, dst, y2).reshape(NC, ACCR, D)

    Wlp = jnp.pad(Wl, ((0, 0), (0, D - 6)))
    blp = jnp.pad(bl, (0, D - 6)).reshape(1, D)
    batch_b = jnp.broadcast_to(batch[:, None], (N, D))
    P = _tc_final(a2, a2, y2, dis_b, b2.reshape(1, D), Wlp, blp, batch_b)

    sums = P[:G, :6]
    counts = P[:G, 6]
    return sums / jnp.clip(counts, 1.0)[:, None]
